# Initial kernel scaffold; baseline (speedup 1.0000x reference)
import functools

import jax
import jax.numpy as jnp
from jax import lax
from jax.experimental import pallas as pl
from jax.experimental.pallas import tpu as pltpu
from jax.experimental.pallas import tpu_sc as plsc


def _sc_info():
    try:
        info = plsc.get_sparse_core_info()
        return info.num_cores, info.num_subcores
    except Exception:
        return 2, 16


_GCHUNK = 128


def _sc_gather(table, idx2d, n_rows, d):
    nc, ns = _sc_info()
    nw = nc * ns
    chunks_total = n_rows // _GCHUNK
    chunks_per_w = chunks_total // nw
    rows_per_w = chunks_per_w * _GCHUNK

    mesh = plsc.VectorSubcoreMesh(core_axis_name="c", subcore_axis_name="s")

    @functools.partial(
        pl.kernel,
        out_type=jax.ShapeDtypeStruct((n_rows, d), jnp.float32),
        mesh=mesh,
        scratch_types=[
            pltpu.VMEM((chunks_per_w, _GCHUNK), jnp.int32),
            pltpu.VMEM((_GCHUNK, d), jnp.float32),
            pltpu.SemaphoreType.DMA,
        ],
    )
    def gather_kernel(table_hbm, idx_hbm, out_hbm, idx_v, rows_v, sem):
        wid = lax.axis_index("s") * nc + lax.axis_index("c")
        cbase = wid * chunks_per_w
        rbase = wid * rows_per_w
        pltpu.sync_copy(idx_hbm.at[pl.ds(cbase, chunks_per_w)], idx_v)

        def body(j, carry):
            pltpu.async_copy(table_hbm.at[idx_v.at[j]], rows_v, sem).wait()
            pltpu.sync_copy(rows_v, out_hbm.at[pl.ds(rbase + j * _GCHUNK, _GCHUNK)])
            return carry

        lax.fori_loop(0, chunks_per_w, body, 0)

    return gather_kernel(table, idx2d)


def _tc_mlp(flat, W1, b1r, W2r, b2r):
    b_, k = flat.shape
    h = W1.shape[1]
    bm = 1024

    def body(x_ref, w1_ref, b1_ref, w2_ref, b2_ref, o_ref):
        x = x_ref[...]
        hh = jnp.maximum(
            jnp.dot(x, w1_ref[...], preferred_element_type=jnp.float32)
            + b1_ref[...],
            0.0,
        )
        o_ref[...] = jnp.sum(hh * w2_ref[...], axis=1, keepdims=True) + b2_ref[...]

    return pl.pallas_call(
        body,
        grid=(b_ // bm,),
        in_specs=[
            pl.BlockSpec((bm, k), lambda i: (i, 0)),
            pl.BlockSpec((k, h), lambda i: (0, 0)),
            pl.BlockSpec((1, h), lambda i: (0, 0)),
            pl.BlockSpec((1, h), lambda i: (0, 0)),
            pl.BlockSpec((1, 1), lambda i: (0, 0)),
        ],
        out_specs=pl.BlockSpec((bm, 1), lambda i: (i, 0)),
        out_shape=jax.ShapeDtypeStruct((b_, 1), jnp.float32),
    )(flat, W1, b1r, W2r, b2r)


def kernel(X, table, W1, b1, W2, b2):
    b_, f = X.shape
    v, d = table.shape
    h = W1.shape[1]
    n_rows = b_ * f

    idx2d = X.reshape(n_rows // _GCHUNK, _GCHUNK)
    rows = _sc_gather(table, idx2d, n_rows, d)
    flat = rows.reshape(b_, f * d)
    return _tc_mlp(flat, W1, b1.reshape(1, h), W2.reshape(1, h), b2.reshape(1, 1))


# speedup vs baseline: 15.4160x; 15.4160x over previous
import functools

import jax
import jax.numpy as jnp
from jax import lax
from jax.experimental import pallas as pl
from jax.experimental.pallas import tpu as pltpu
from jax.experimental.pallas import tpu_sc as plsc


def _sc_info():
    try:
        info = plsc.get_sparse_core_info()
        return info.num_cores, info.num_subcores
    except Exception:
        return 2, 16


_GCHUNK = 128


def _sc_gather(table, idx2d, n_rows, d):
    nc, ns = _sc_info()
    nw = nc * ns
    chunks_total = n_rows // _GCHUNK
    chunks_per_w = chunks_total // nw
    rows_per_w = chunks_per_w * _GCHUNK

    mesh = plsc.VectorSubcoreMesh(core_axis_name="c", subcore_axis_name="s")

    @functools.partial(
        pl.kernel,
        out_type=jax.ShapeDtypeStruct((n_rows, d), jnp.float32),
        mesh=mesh,
        scratch_types=[
            pltpu.VMEM((chunks_per_w, _GCHUNK), jnp.int32),
            pltpu.VMEM((_GCHUNK, d), jnp.float32),
            pltpu.SemaphoreType.DMA,
        ],
        compiler_params=pltpu.CompilerParams(use_tc_tiling_on_sc=False),
    )
    def gather_kernel(table_hbm, idx_hbm, out_hbm, idx_v, rows_v, sem):
        wid = lax.axis_index("s") * nc + lax.axis_index("c")
        cbase = wid * chunks_per_w
        rbase = wid * rows_per_w
        pltpu.sync_copy(idx_hbm.at[pl.ds(cbase, chunks_per_w)], idx_v)

        def body(j, carry):
            pltpu.async_copy(table_hbm.at[idx_v.at[j]], rows_v, sem).wait()
            pltpu.sync_copy(rows_v, out_hbm.at[pl.ds(rbase + j * _GCHUNK, _GCHUNK)])
            return carry

        lax.fori_loop(0, chunks_per_w, body, 0)

    return gather_kernel(table, idx2d)


def _tc_mlp(flat, W1, b1r, W2r, b2r):
    b_, k = flat.shape
    h = W1.shape[1]
    bm = 1024

    def body(x_ref, w1_ref, b1_ref, w2_ref, b2_ref, o_ref):
        x = x_ref[...]
        hh = jnp.maximum(
            jnp.dot(x, w1_ref[...], preferred_element_type=jnp.float32)
            + b1_ref[...],
            0.0,
        )
        o_ref[...] = jnp.sum(hh * w2_ref[...], axis=1, keepdims=True) + b2_ref[...]

    return pl.pallas_call(
        body,
        grid=(b_ // bm,),
        in_specs=[
            pl.BlockSpec((bm, k), lambda i: (i, 0)),
            pl.BlockSpec((k, h), lambda i: (0, 0)),
            pl.BlockSpec((1, h), lambda i: (0, 0)),
            pl.BlockSpec((1, h), lambda i: (0, 0)),
            pl.BlockSpec((1, 1), lambda i: (0, 0)),
        ],
        out_specs=pl.BlockSpec((bm, 1), lambda i: (i, 0)),
        out_shape=jax.ShapeDtypeStruct((b_, 1), jnp.float32),
    )(flat, W1, b1r, W2r, b2r)


def kernel(X, table, W1, b1, W2, b2):
    b_, f = X.shape
    v, d = table.shape
    h = W1.shape[1]
    n_rows = b_ * f

    idx2d = X.reshape(n_rows // _GCHUNK, _GCHUNK)
    rows = _sc_gather(table, idx2d, n_rows, d)
    flat = rows.reshape(b_, f * d)
    return _tc_mlp(flat, W1, b1.reshape(1, h), W2.reshape(1, h), b2.reshape(1, 1))
